# Initial kernel scaffold; baseline (speedup 1.0000x reference)
import jax
import jax.numpy as jnp
from jax.experimental import pallas as pl


def kernel(x, edge_index, edge_attr, W1, b1, W2, b2, root, conv1_bias, gcn_W, gcn_b, lin_W, lin_b):
    raise NotImplementedError("write your pallas kernel here")


# speedup vs baseline: 4.9640x; 4.9640x over previous
import functools

import jax
import jax.numpy as jnp
from jax import lax
from jax.experimental import pallas as pl
from jax.experimental.pallas import tpu as pltpu
from jax.experimental.pallas import tpu_sc as plsc

N = 10000
E = 160000
IN_C = 128
HID = 8
H2 = IN_C * HID
W = 16

NW = 32
NC = 2
CHUNK = 128
NUM_CHUNKS = E // CHUNK
ITERS = (NUM_CHUNKS + NW - 1) // NW
ROWS_PER_TILE = N // 16


def _mesh():
    return plsc.VectorSubcoreMesh(core_axis_name="c", subcore_axis_name="s")


_SC_PARAMS = pltpu.CompilerParams(use_tc_tiling_on_sc=False,
                                  needs_layout_passes=False)


def _sc_gather_x(x, src):
    @functools.partial(
        pl.kernel,
        mesh=_mesh(),
        compiler_params=_SC_PARAMS,
        out_type=jax.ShapeDtypeStruct((E, IN_C), jnp.float32),
        scratch_types=[
            pltpu.VMEM((CHUNK,), jnp.int32),
            pltpu.VMEM((CHUNK, IN_C), jnp.float32),
            pltpu.SemaphoreType.DMA,
        ],
    )
    def k(x_hbm, src_hbm, out_hbm, idx_v, rows_v, sem):
        wid = lax.axis_index("s") * NC + lax.axis_index("c")

        def body(i, carry):
            c = wid + i * NW

            @pl.when(c < NUM_CHUNKS)
            def _():
                base = c * CHUNK
                pltpu.sync_copy(src_hbm.at[pl.ds(base, CHUNK)], idx_v)
                pltpu.async_copy(x_hbm.at[idx_v], rows_v, sem).wait()
                pltpu.sync_copy(rows_v, out_hbm.at[pl.ds(base, CHUNK)])

            return carry

        lax.fori_loop(0, ITERS, body, 0)

    return k(x, src)


TE = 256
GRID = E // TE
RPP = N // GRID


def _tc_nnconv(ea, xj, W1, b1r, W2p, b2pr, x, root, c1br):
    def body(ea_ref, xj_ref, w1_ref, b1_ref, w2_ref, b2_ref, x_ref, root_ref,
             c1b_ref, msg_ref, xroot_ref):
        r = jnp.dot(ea_ref[...], w1_ref[...], preferred_element_type=jnp.float32)
        r = jnp.maximum(r + b1_ref[...], 0.0).astype(jnp.bfloat16)
        h = jnp.dot(r, w2_ref[...], preferred_element_type=jnp.float32)
        h = h + b2_ref[...]
        xjv = xj_ref[...]
        cols = [jnp.sum(h[:, o * IN_C:(o + 1) * IN_C] * xjv, axis=1, keepdims=True)
                for o in range(HID)]
        cols.append(jnp.ones((TE, 1), jnp.float32))
        cols.append(jnp.zeros((TE, W - HID - 1), jnp.float32))
        msg_ref[...] = jnp.concatenate(cols, axis=1)
        xroot_ref[...] = (
            jnp.dot(x_ref[...], root_ref[...], preferred_element_type=jnp.float32)
            + c1b_ref[...])

    return pl.pallas_call(
        body,
        grid=(GRID,),
        in_specs=[
            pl.BlockSpec((TE, 16), lambda i: (i, 0)),
            pl.BlockSpec((TE, IN_C), lambda i: (i, 0)),
            pl.BlockSpec((16, H2), lambda i: (0, 0)),
            pl.BlockSpec((1, H2), lambda i: (0, 0)),
            pl.BlockSpec((H2, H2), lambda i: (0, 0)),
            pl.BlockSpec((1, H2), lambda i: (0, 0)),
            pl.BlockSpec((RPP, IN_C), lambda i: (i, 0)),
            pl.BlockSpec((IN_C, HID), lambda i: (0, 0)),
            pl.BlockSpec((1, HID), lambda i: (0, 0)),
        ],
        out_specs=[
            pl.BlockSpec((TE, W), lambda i: (i, 0)),
            pl.BlockSpec((RPP, HID), lambda i: (i, 0)),
        ],
        out_shape=[
            jax.ShapeDtypeStruct((E, W), jnp.float32),
            jax.ShapeDtypeStruct((N, HID), jnp.float32),
        ],
    )(ea, xj, W1, b1r, W2p, b2pr, x, root, c1br)


def _zero_spmem(z_v, acc_sh, sid):
    def zb(i, carry):
        for j in range(W // 16):
            z_v[i, pl.ds(j * 16, 16)] = jnp.zeros((16,), jnp.float32)
        return carry

    lax.fori_loop(0, ROWS_PER_TILE, zb, 0)
    pltpu.sync_copy(z_v, acc_sh.at[pl.ds(sid * ROWS_PER_TILE, ROWS_PER_TILE)])


def _sc_scatter_rows(vals, dst):
    @functools.partial(
        pl.kernel,
        mesh=_mesh(),
        compiler_params=_SC_PARAMS,
        out_type=jax.ShapeDtypeStruct((NC, N, W), jnp.float32),
        scratch_types=[
            pltpu.VMEM((CHUNK,), jnp.int32),
            pltpu.VMEM((CHUNK, W), jnp.float32),
            pltpu.VMEM((ROWS_PER_TILE, W), jnp.float32),
            pltpu.VMEM_SHARED((N, W), jnp.float32),
        ],
    )
    def k(vals_hbm, dst_hbm, out_hbm, idx_v, m_v, z_v, acc_sh):
        cid = lax.axis_index("c")
        sid = lax.axis_index("s")
        wid = sid * NC + cid

        _zero_spmem(z_v, acc_sh, sid)
        plsc.subcore_barrier()

        def body(i, carry):
            c = wid + i * NW

            @pl.when(c < NUM_CHUNKS)
            def _():
                base = c * CHUNK
                pltpu.sync_copy(dst_hbm.at[pl.ds(base, CHUNK)], idx_v)
                pltpu.sync_copy(vals_hbm.at[pl.ds(base, CHUNK)], m_v)
                pltpu.sync_copy(m_v, acc_sh.at[idx_v], add=True)

            return carry

        lax.fori_loop(0, ITERS, body, 0)
        plsc.subcore_barrier()

        @pl.when(sid == 0)
        def _():
            pltpu.sync_copy(acc_sh, out_hbm.at[cid])

    return k(vals, dst)


def _tc_node1(p, xroot, gcn_W):
    def body(p_ref, xr_ref, gw_ref, out_ref):
        agg = p_ref[0, :, 0:HID] + p_ref[1, :, 0:HID]
        deg = p_ref[0, :, HID:HID + 1] + p_ref[1, :, HID:HID + 1] + 1.0
        h1 = jnp.maximum(agg + xr_ref[...], 0.0)
        dinv = lax.rsqrt(deg)
        xw = jnp.dot(h1, gw_ref[...], preferred_element_type=jnp.float32)
        xwd = xw * dinv
        out_ref[...] = jnp.concatenate(
            [xwd, dinv, jnp.zeros((N, W - HID - 1), jnp.float32)], axis=1)

    return pl.pallas_call(
        body,
        out_shape=jax.ShapeDtypeStruct((N, W), jnp.float32),
    )(p, xroot, gcn_W)


def _sc_gather_scatter(xwd, src, dst):
    @functools.partial(
        pl.kernel,
        mesh=_mesh(),
        compiler_params=_SC_PARAMS,
        out_type=jax.ShapeDtypeStruct((NC, N, W), jnp.float32),
        scratch_types=[
            pltpu.VMEM((CHUNK,), jnp.int32),
            pltpu.VMEM((CHUNK,), jnp.int32),
            pltpu.VMEM((CHUNK, W), jnp.float32),
            pltpu.VMEM((ROWS_PER_TILE, W), jnp.float32),
            pltpu.VMEM_SHARED((N, W), jnp.float32),
            pltpu.SemaphoreType.DMA,
        ],
    )
    def k(xwd_hbm, src_hbm, dst_hbm, out_hbm, is_v, id_v, rows_v, z_v, acc_sh,
          sem):
        cid = lax.axis_index("c")
        sid = lax.axis_index("s")
        wid = sid * NC + cid

        _zero_spmem(z_v, acc_sh, sid)
        plsc.subcore_barrier()

        def body(i, carry):
            c = wid + i * NW

            @pl.when(c < NUM_CHUNKS)
            def _():
                base = c * CHUNK
                pltpu.sync_copy(src_hbm.at[pl.ds(base, CHUNK)], is_v)
                pltpu.sync_copy(dst_hbm.at[pl.ds(base, CHUNK)], id_v)
                pltpu.async_copy(xwd_hbm.at[is_v], rows_v, sem).wait()
                pltpu.sync_copy(rows_v, acc_sh.at[id_v], add=True)

            return carry

        lax.fori_loop(0, ITERS, body, 0)
        plsc.subcore_barrier()

        @pl.when(sid == 0)
        def _():
            pltpu.sync_copy(acc_sh, out_hbm.at[cid])

    return k(xwd, src, dst)


def _tc_node2(p2, xwd, gcn_br, lin_WT, lin_br):
    def body(p_ref, xwd_ref, gb_ref, lw_ref, lb_ref, out_ref):
        dinv = xwd_ref[:, HID:HID + 1]
        agg2 = p_ref[0, :, 0:HID] + p_ref[1, :, 0:HID] + xwd_ref[:, 0:HID]
        h2 = agg2 * dinv + gb_ref[...]
        wu = lw_ref[:, 0:HID]
        wv = lw_ref[:, HID:2 * HID]
        u = jnp.sum(h2 * wu, axis=1, keepdims=True) + lb_ref[...]
        v = jnp.sum(h2 * wv, axis=1, keepdims=True)
        out_ref[...] = jnp.concatenate(
            [u, v, jnp.zeros((N, 14), jnp.float32)], axis=1)

    return pl.pallas_call(
        body,
        out_shape=jax.ShapeDtypeStruct((N, 16), jnp.float32),
    )(p2, xwd, gcn_br, lin_WT, lin_br)


def _sc_edge_scores(u, v, src, dst):
    @functools.partial(
        pl.kernel,
        mesh=_mesh(),
        compiler_params=_SC_PARAMS,
        out_type=jax.ShapeDtypeStruct((E,), jnp.float32),
        scratch_types=[
            pltpu.VMEM((N,), jnp.float32),
            pltpu.VMEM((N,), jnp.float32),
            pltpu.VMEM((CHUNK,), jnp.int32),
            pltpu.VMEM((CHUNK,), jnp.int32),
            pltpu.VMEM((CHUNK,), jnp.float32),
        ],
    )
    def k(u_hbm, v_hbm, src_hbm, dst_hbm, out_hbm, u_v, v_v, is_v, id_v, o_v):
        wid = lax.axis_index("s") * NC + lax.axis_index("c")
        pltpu.sync_copy(u_hbm, u_v)
        pltpu.sync_copy(v_hbm, v_v)

        def body(i, carry):
            c = wid + i * NW

            @pl.when(c < NUM_CHUNKS)
            def _():
                base = c * CHUNK
                pltpu.sync_copy(src_hbm.at[pl.ds(base, CHUNK)], is_v)
                pltpu.sync_copy(dst_hbm.at[pl.ds(base, CHUNK)], id_v)
                for g in range(CHUNK // 16):
                    s16 = is_v[pl.ds(g * 16, 16)]
                    d16 = id_v[pl.ds(g * 16, 16)]
                    us = plsc.load_gather(u_v, [s16])
                    vd = plsc.load_gather(v_v, [d16])
                    o_v[pl.ds(g * 16, 16)] = us + vd
                pltpu.sync_copy(o_v, out_hbm.at[pl.ds(base, CHUNK)])

            return carry

        lax.fori_loop(0, ITERS, body, 0)

    return k(u, v, src, dst)


def _tc_sigmoid(scores2d):
    def body(s_ref, out_ref):
        out_ref[...] = jax.nn.sigmoid(s_ref[...])

    return pl.pallas_call(
        body,
        out_shape=jax.ShapeDtypeStruct(scores2d.shape, jnp.float32),
    )(scores2d)


def kernel(x, edge_index, edge_attr, W1, b1, W2, b2, root, conv1_bias,
           gcn_W, gcn_b, lin_W, lin_b):
    src = edge_index[0]
    dst = edge_index[1]
    W2p = (W2.reshape(H2, IN_C, HID).transpose(0, 2, 1).reshape(H2, H2)
           .astype(jnp.bfloat16))
    b2pr = b2.reshape(IN_C, HID).T.reshape(1, H2)
    b1r = b1.reshape(1, H2)
    c1br = conv1_bias.reshape(1, HID)
    gcn_br = gcn_b.reshape(1, HID)
    lin_WT = lin_W.reshape(2 * HID, 1).T
    lin_br = lin_b.reshape(1, 1)

    xj = _sc_gather_x(x, src)
    msgc, xroot = _tc_nnconv(edge_attr, xj, W1, b1r, W2p, b2pr, x, root, c1br)
    p = _sc_scatter_rows(msgc, dst)
    xwd = _tc_node1(p, xroot, gcn_W)
    p2 = _sc_gather_scatter(xwd, src, dst)
    uv = _tc_node2(p2, xwd, gcn_br, lin_WT, lin_br)
    u = uv[:, 0]
    v = uv[:, 1]
    scores = _sc_edge_scores(u, v, src, dst)
    return _tc_sigmoid(scores.reshape(E // 128, 128)).reshape(E)


# speedup vs baseline: 5.7409x; 1.1565x over previous
import functools

import jax
import jax.numpy as jnp
from jax import lax
from jax.experimental import pallas as pl
from jax.experimental.pallas import tpu as pltpu
from jax.experimental.pallas import tpu_sc as plsc

N = 10000
E = 160000
IN_C = 128
HID = 8
H2 = IN_C * HID
W = 16

NW = 32
NC = 2
CHUNK = 128
NUM_CHUNKS = E // CHUNK
ITERS = (NUM_CHUNKS + NW - 1) // NW
ROWS_PER_TILE = N // 16
PIPE = 4


def _mesh():
    return plsc.VectorSubcoreMesh(core_axis_name="c", subcore_axis_name="s")


_SC_PARAMS = pltpu.CompilerParams(use_tc_tiling_on_sc=False,
                                  needs_layout_passes=False)


def _sc_gather_x(x, src):
    @functools.partial(
        pl.kernel,
        mesh=_mesh(),
        compiler_params=_SC_PARAMS,
        out_type=jax.ShapeDtypeStruct((E, IN_C), jnp.float32),
        scratch_types=[
            pltpu.VMEM((PIPE, CHUNK), jnp.int32),
            pltpu.VMEM((PIPE, CHUNK, IN_C), jnp.float32),
            pltpu.SemaphoreType.DMA,
            pltpu.SemaphoreType.DMA,
            pltpu.SemaphoreType.DMA,
        ],
    )
    def k(x_hbm, src_hbm, out_hbm, idx_v, rows_v, isem, gsem, osem):
        wid = lax.axis_index("s") * NC + lax.axis_index("c")

        def body(i, carry):
            def quad(stage):
                for b in range(PIPE):
                    c = wid + (i * PIPE + b) * NW

                    @pl.when(c < NUM_CHUNKS)
                    def _():
                        stage(b, c)

            quad(lambda b, c: pltpu.async_copy(
                src_hbm.at[pl.ds(c * CHUNK, CHUNK)], idx_v.at[b], isem))
            quad(lambda b, c: pltpu.make_async_copy(
                src_hbm.at[pl.ds(c * CHUNK, CHUNK)], idx_v.at[b], isem).wait())
            quad(lambda b, c: pltpu.async_copy(
                x_hbm.at[idx_v.at[b]], rows_v.at[b], gsem))
            quad(lambda b, c: pltpu.make_async_copy(
                x_hbm.at[idx_v.at[b]], rows_v.at[b], gsem).wait())
            quad(lambda b, c: pltpu.async_copy(
                rows_v.at[b], out_hbm.at[pl.ds(c * CHUNK, CHUNK)], osem))
            quad(lambda b, c: pltpu.make_async_copy(
                rows_v.at[b], out_hbm.at[pl.ds(c * CHUNK, CHUNK)], osem).wait())
            return carry

        lax.fori_loop(0, (ITERS + PIPE - 1) // PIPE, body, 0)

    return k(x, src)


TE = 640
GRID = E // TE
RPP = N // GRID


def _tc_nnconv(ea, xj, W1, b1r, W2p, b2pr, x, root, c1br):
    def body(ea_ref, xj_ref, w1_ref, b1_ref, w2_ref, b2_ref, x_ref, root_ref,
             c1b_ref, msg_ref, xroot_ref):
        r = jnp.dot(ea_ref[...], w1_ref[...], preferred_element_type=jnp.float32)
        r = jnp.maximum(r + b1_ref[...], 0.0).astype(jnp.bfloat16)
        h = jnp.dot(r, w2_ref[...], preferred_element_type=jnp.float32)
        h = h + b2_ref[...]
        xjv = xj_ref[...]
        cols = [jnp.sum(h[:, o * IN_C:(o + 1) * IN_C] * xjv, axis=1, keepdims=True)
                for o in range(HID)]
        cols.append(jnp.ones((TE, 1), jnp.float32))
        cols.append(jnp.zeros((TE, W - HID - 1), jnp.float32))
        msg_ref[...] = jnp.concatenate(cols, axis=1)
        xroot_ref[...] = (
            jnp.dot(x_ref[...], root_ref[...], preferred_element_type=jnp.float32)
            + c1b_ref[...])

    return pl.pallas_call(
        body,
        grid=(GRID,),
        in_specs=[
            pl.BlockSpec((TE, 16), lambda i: (i, 0)),
            pl.BlockSpec((TE, IN_C), lambda i: (i, 0)),
            pl.BlockSpec((16, H2), lambda i: (0, 0)),
            pl.BlockSpec((1, H2), lambda i: (0, 0)),
            pl.BlockSpec((H2, H2), lambda i: (0, 0)),
            pl.BlockSpec((1, H2), lambda i: (0, 0)),
            pl.BlockSpec((RPP, IN_C), lambda i: (i, 0)),
            pl.BlockSpec((IN_C, HID), lambda i: (0, 0)),
            pl.BlockSpec((1, HID), lambda i: (0, 0)),
        ],
        out_specs=[
            pl.BlockSpec((TE, W), lambda i: (i, 0)),
            pl.BlockSpec((RPP, HID), lambda i: (i, 0)),
        ],
        out_shape=[
            jax.ShapeDtypeStruct((E, W), jnp.float32),
            jax.ShapeDtypeStruct((N, HID), jnp.float32),
        ],
    )(ea, xj, W1, b1r, W2p, b2pr, x, root, c1br)


def _zero_spmem(z_v, acc_sh, sid):
    def zb(i, carry):
        for j in range(W // 16):
            z_v[i, pl.ds(j * 16, 16)] = jnp.zeros((16,), jnp.float32)
        return carry

    lax.fori_loop(0, ROWS_PER_TILE, zb, 0)
    pltpu.sync_copy(z_v, acc_sh.at[pl.ds(sid * ROWS_PER_TILE, ROWS_PER_TILE)])


def _sc_scatter_rows(vals, dst):
    @functools.partial(
        pl.kernel,
        mesh=_mesh(),
        compiler_params=_SC_PARAMS,
        out_type=jax.ShapeDtypeStruct((NC, N, W), jnp.float32),
        scratch_types=[
            pltpu.VMEM((CHUNK,), jnp.int32),
            pltpu.VMEM((CHUNK, W), jnp.float32),
            pltpu.VMEM((ROWS_PER_TILE, W), jnp.float32),
            pltpu.VMEM_SHARED((N, W), jnp.float32),
        ],
    )
    def k(vals_hbm, dst_hbm, out_hbm, idx_v, m_v, z_v, acc_sh):
        cid = lax.axis_index("c")
        sid = lax.axis_index("s")
        wid = sid * NC + cid

        _zero_spmem(z_v, acc_sh, sid)
        plsc.subcore_barrier()

        def body(i, carry):
            c = wid + i * NW

            @pl.when(c < NUM_CHUNKS)
            def _():
                base = c * CHUNK
                pltpu.sync_copy(dst_hbm.at[pl.ds(base, CHUNK)], idx_v)
                pltpu.sync_copy(vals_hbm.at[pl.ds(base, CHUNK)], m_v)
                pltpu.sync_copy(m_v, acc_sh.at[idx_v], add=True)

            return carry

        lax.fori_loop(0, ITERS, body, 0)
        plsc.subcore_barrier()

        @pl.when(sid == 0)
        def _():
            pltpu.sync_copy(acc_sh, out_hbm.at[cid])

    return k(vals, dst)


def _tc_node1(p, xroot, gcn_W):
    def body(p_ref, xr_ref, gw_ref, out_ref):
        agg = p_ref[0, :, 0:HID] + p_ref[1, :, 0:HID]
        deg = p_ref[0, :, HID:HID + 1] + p_ref[1, :, HID:HID + 1] + 1.0
        h1 = jnp.maximum(agg + xr_ref[...], 0.0)
        dinv = lax.rsqrt(deg)
        xw = jnp.dot(h1, gw_ref[...], preferred_element_type=jnp.float32)
        xwd = xw * dinv
        out_ref[...] = jnp.concatenate(
            [xwd, dinv, jnp.zeros((N, W - HID - 1), jnp.float32)], axis=1)

    return pl.pallas_call(
        body,
        out_shape=jax.ShapeDtypeStruct((N, W), jnp.float32),
    )(p, xroot, gcn_W)


def _sc_gather_scatter(xwd, src, dst):
    @functools.partial(
        pl.kernel,
        mesh=_mesh(),
        compiler_params=_SC_PARAMS,
        out_type=jax.ShapeDtypeStruct((NC, N, W), jnp.float32),
        scratch_types=[
            pltpu.VMEM((CHUNK,), jnp.int32),
            pltpu.VMEM((CHUNK,), jnp.int32),
            pltpu.VMEM((CHUNK, W), jnp.float32),
            pltpu.VMEM((ROWS_PER_TILE, W), jnp.float32),
            pltpu.VMEM_SHARED((N, W), jnp.float32),
            pltpu.SemaphoreType.DMA,
        ],
    )
    def k(xwd_hbm, src_hbm, dst_hbm, out_hbm, is_v, id_v, rows_v, z_v, acc_sh,
          sem):
        cid = lax.axis_index("c")
        sid = lax.axis_index("s")
        wid = sid * NC + cid

        _zero_spmem(z_v, acc_sh, sid)
        plsc.subcore_barrier()

        def body(i, carry):
            c = wid + i * NW

            @pl.when(c < NUM_CHUNKS)
            def _():
                base = c * CHUNK
                pltpu.sync_copy(src_hbm.at[pl.ds(base, CHUNK)], is_v)
                pltpu.sync_copy(dst_hbm.at[pl.ds(base, CHUNK)], id_v)
                pltpu.async_copy(xwd_hbm.at[is_v], rows_v, sem).wait()
                pltpu.sync_copy(rows_v, acc_sh.at[id_v], add=True)

            return carry

        lax.fori_loop(0, ITERS, body, 0)
        plsc.subcore_barrier()

        @pl.when(sid == 0)
        def _():
            pltpu.sync_copy(acc_sh, out_hbm.at[cid])

    return k(xwd, src, dst)


def _tc_node2(p2, xwd, gcn_br, lin_WT, lin_br):
    def body(p_ref, xwd_ref, gb_ref, lw_ref, lb_ref, out_ref):
        dinv = xwd_ref[:, HID:HID + 1]
        agg2 = p_ref[0, :, 0:HID] + p_ref[1, :, 0:HID] + xwd_ref[:, 0:HID]
        h2 = agg2 * dinv + gb_ref[...]
        wu = lw_ref[:, 0:HID]
        wv = lw_ref[:, HID:2 * HID]
        u = jnp.sum(h2 * wu, axis=1, keepdims=True) + lb_ref[...]
        v = jnp.sum(h2 * wv, axis=1, keepdims=True)
        out_ref[...] = jnp.concatenate(
            [u, v, jnp.zeros((N, 14), jnp.float32)], axis=1)

    return pl.pallas_call(
        body,
        out_shape=jax.ShapeDtypeStruct((N, 16), jnp.float32),
    )(p2, xwd, gcn_br, lin_WT, lin_br)


def _sc_edge_scores(u, v, src, dst):
    @functools.partial(
        pl.kernel,
        mesh=_mesh(),
        compiler_params=_SC_PARAMS,
        out_type=jax.ShapeDtypeStruct((E,), jnp.float32),
        scratch_types=[
            pltpu.VMEM((N,), jnp.float32),
            pltpu.VMEM((N,), jnp.float32),
            pltpu.VMEM((CHUNK,), jnp.int32),
            pltpu.VMEM((CHUNK,), jnp.int32),
            pltpu.VMEM((CHUNK,), jnp.float32),
        ],
    )
    def k(u_hbm, v_hbm, src_hbm, dst_hbm, out_hbm, u_v, v_v, is_v, id_v, o_v):
        wid = lax.axis_index("s") * NC + lax.axis_index("c")
        pltpu.sync_copy(u_hbm, u_v)
        pltpu.sync_copy(v_hbm, v_v)

        def body(i, carry):
            c = wid + i * NW

            @pl.when(c < NUM_CHUNKS)
            def _():
                base = c * CHUNK
                pltpu.sync_copy(src_hbm.at[pl.ds(base, CHUNK)], is_v)
                pltpu.sync_copy(dst_hbm.at[pl.ds(base, CHUNK)], id_v)
                for g in range(CHUNK // 16):
                    s16 = is_v[pl.ds(g * 16, 16)]
                    d16 = id_v[pl.ds(g * 16, 16)]
                    us = plsc.load_gather(u_v, [s16])
                    vd = plsc.load_gather(v_v, [d16])
                    o_v[pl.ds(g * 16, 16)] = us + vd
                pltpu.sync_copy(o_v, out_hbm.at[pl.ds(base, CHUNK)])

            return carry

        lax.fori_loop(0, ITERS, body, 0)

    return k(u, v, src, dst)


def _tc_sigmoid(scores2d):
    def body(s_ref, out_ref):
        out_ref[...] = jax.nn.sigmoid(s_ref[...])

    return pl.pallas_call(
        body,
        out_shape=jax.ShapeDtypeStruct(scores2d.shape, jnp.float32),
    )(scores2d)


def kernel(x, edge_index, edge_attr, W1, b1, W2, b2, root, conv1_bias,
           gcn_W, gcn_b, lin_W, lin_b):
    src = edge_index[0]
    dst = edge_index[1]
    W2p = (W2.reshape(H2, IN_C, HID).transpose(0, 2, 1).reshape(H2, H2)
           .astype(jnp.bfloat16))
    b2pr = b2.reshape(IN_C, HID).T.reshape(1, H2)
    b1r = b1.reshape(1, H2)
    c1br = conv1_bias.reshape(1, HID)
    gcn_br = gcn_b.reshape(1, HID)
    lin_WT = lin_W.reshape(2 * HID, 1).T
    lin_br = lin_b.reshape(1, 1)

    xj = _sc_gather_x(x, src)
    msgc, xroot = _tc_nnconv(edge_attr, xj, W1, b1r, W2p, b2pr, x, root, c1br)
    p = _sc_scatter_rows(msgc, dst)
    xwd = _tc_node1(p, xroot, gcn_W)
    p2 = _sc_gather_scatter(xwd, src, dst)
    uv = _tc_node2(p2, xwd, gcn_br, lin_WT, lin_br)
    u = uv[:, 0]
    v = uv[:, 1]
    scores = _sc_edge_scores(u, v, src, dst)
    return _tc_sigmoid(scores.reshape(E // 128, 128)).reshape(E)


# speedup vs baseline: 6.5588x; 1.1425x over previous
import functools

import jax
import jax.numpy as jnp
from jax import lax
from jax.experimental import pallas as pl
from jax.experimental.pallas import tpu as pltpu
from jax.experimental.pallas import tpu_sc as plsc

N = 10000
E = 160000
IN_C = 128
HID = 8
H2 = IN_C * HID
W = 16

NW = 32
NC = 2
CHUNK = 128
NUM_CHUNKS = E // CHUNK
ITERS = (NUM_CHUNKS + NW - 1) // NW
ROWS_PER_TILE = N // 16
PIPE = 4


def _mesh():
    return plsc.VectorSubcoreMesh(core_axis_name="c", subcore_axis_name="s")


_SC_PARAMS = pltpu.CompilerParams(use_tc_tiling_on_sc=False,
                                  needs_layout_passes=False)


def _sc_gather_x(x, src):
    @functools.partial(
        pl.kernel,
        mesh=_mesh(),
        compiler_params=_SC_PARAMS,
        out_type=jax.ShapeDtypeStruct((E, IN_C), jnp.float32),
        scratch_types=[
            pltpu.VMEM((PIPE, CHUNK), jnp.int32),
            pltpu.VMEM((PIPE, CHUNK, IN_C), jnp.float32),
            pltpu.SemaphoreType.DMA,
            pltpu.SemaphoreType.DMA,
            pltpu.SemaphoreType.DMA,
        ],
    )
    def k(x_hbm, src_hbm, out_hbm, idx_v, rows_v, isem, gsem, osem):
        wid = lax.axis_index("s") * NC + lax.axis_index("c")

        def body(i, carry):
            def quad(stage):
                for b in range(PIPE):
                    c = wid + (i * PIPE + b) * NW

                    @pl.when(c < NUM_CHUNKS)
                    def _():
                        stage(b, c)

            quad(lambda b, c: pltpu.async_copy(
                src_hbm.at[pl.ds(c * CHUNK, CHUNK)], idx_v.at[b], isem))
            quad(lambda b, c: pltpu.make_async_copy(
                src_hbm.at[pl.ds(c * CHUNK, CHUNK)], idx_v.at[b], isem).wait())
            quad(lambda b, c: pltpu.async_copy(
                x_hbm.at[idx_v.at[b]], rows_v.at[b], gsem))
            quad(lambda b, c: pltpu.make_async_copy(
                x_hbm.at[idx_v.at[b]], rows_v.at[b], gsem).wait())
            quad(lambda b, c: pltpu.async_copy(
                rows_v.at[b], out_hbm.at[pl.ds(c * CHUNK, CHUNK)], osem))
            quad(lambda b, c: pltpu.make_async_copy(
                rows_v.at[b], out_hbm.at[pl.ds(c * CHUNK, CHUNK)], osem).wait())
            return carry

        lax.fori_loop(0, (ITERS + PIPE - 1) // PIPE, body, 0)

    return k(x, src)


TE = 640
GRID = E // TE
RPP = N // GRID


def _tc_nnconv(ea, xj, W1, b1r, W2p, b2pr, x, root, c1br):
    def body(ea_ref, xj_ref, w1_ref, b1_ref, w2_ref, b2_ref, x_ref, root_ref,
             c1b_ref, msg_ref, xroot_ref):
        r = jnp.dot(ea_ref[...], w1_ref[...], preferred_element_type=jnp.float32)
        r = jnp.maximum(r + b1_ref[...], 0.0).astype(jnp.bfloat16)
        h = jnp.dot(r, w2_ref[...], preferred_element_type=jnp.float32)
        h = h + b2_ref[...]
        xjv = xj_ref[...]
        cols = [jnp.sum(h[:, o * IN_C:(o + 1) * IN_C] * xjv, axis=1, keepdims=True)
                for o in range(HID)]
        cols.append(jnp.ones((TE, 1), jnp.float32))
        cols.append(jnp.zeros((TE, W - HID - 1), jnp.float32))
        msg_ref[...] = jnp.concatenate(cols, axis=1)
        xroot_ref[...] = (
            jnp.dot(x_ref[...], root_ref[...], preferred_element_type=jnp.float32)
            + c1b_ref[...])

    return pl.pallas_call(
        body,
        grid=(GRID,),
        in_specs=[
            pl.BlockSpec((TE, 16), lambda i: (i, 0)),
            pl.BlockSpec((TE, IN_C), lambda i: (i, 0)),
            pl.BlockSpec((16, H2), lambda i: (0, 0)),
            pl.BlockSpec((1, H2), lambda i: (0, 0)),
            pl.BlockSpec((H2, H2), lambda i: (0, 0)),
            pl.BlockSpec((1, H2), lambda i: (0, 0)),
            pl.BlockSpec((RPP, IN_C), lambda i: (i, 0)),
            pl.BlockSpec((IN_C, HID), lambda i: (0, 0)),
            pl.BlockSpec((1, HID), lambda i: (0, 0)),
        ],
        out_specs=[
            pl.BlockSpec((TE, W), lambda i: (i, 0)),
            pl.BlockSpec((RPP, HID), lambda i: (i, 0)),
        ],
        out_shape=[
            jax.ShapeDtypeStruct((E, W), jnp.float32),
            jax.ShapeDtypeStruct((N, HID), jnp.float32),
        ],
    )(ea, xj, W1, b1r, W2p, b2pr, x, root, c1br)


def _zero_spmem(z_v, acc_sh, sid):
    def zb(i, carry):
        for j in range(W // 16):
            z_v[i, pl.ds(j * 16, 16)] = jnp.zeros((16,), jnp.float32)
        return carry

    lax.fori_loop(0, ROWS_PER_TILE, zb, 0)
    pltpu.sync_copy(z_v, acc_sh.at[pl.ds(sid * ROWS_PER_TILE, ROWS_PER_TILE)])


def _sc_scatter_rows(vals, dst):
    @functools.partial(
        pl.kernel,
        mesh=_mesh(),
        compiler_params=_SC_PARAMS,
        out_type=jax.ShapeDtypeStruct((NC, N, W), jnp.float32),
        scratch_types=[
            pltpu.VMEM((PIPE, CHUNK), jnp.int32),
            pltpu.VMEM((PIPE, CHUNK, W), jnp.float32),
            pltpu.VMEM((ROWS_PER_TILE, W), jnp.float32),
            pltpu.VMEM_SHARED((N, W), jnp.float32),
            pltpu.SemaphoreType.DMA,
            pltpu.SemaphoreType.DMA,
        ],
    )
    def k(vals_hbm, dst_hbm, out_hbm, idx_v, m_v, z_v, acc_sh, isem, vsem):
        cid = lax.axis_index("c")
        sid = lax.axis_index("s")
        wid = sid * NC + cid

        _zero_spmem(z_v, acc_sh, sid)
        plsc.subcore_barrier()

        def body(i, carry):
            def quad(stage):
                for b in range(PIPE):
                    c = wid + (i * PIPE + b) * NW

                    @pl.when(c < NUM_CHUNKS)
                    def _():
                        stage(b, c * CHUNK)

            quad(lambda b, base: pltpu.async_copy(
                dst_hbm.at[pl.ds(base, CHUNK)], idx_v.at[b], isem))
            quad(lambda b, base: pltpu.async_copy(
                vals_hbm.at[pl.ds(base, CHUNK)], m_v.at[b], vsem))
            quad(lambda b, base: pltpu.make_async_copy(
                dst_hbm.at[pl.ds(base, CHUNK)], idx_v.at[b], isem).wait())
            quad(lambda b, base: pltpu.make_async_copy(
                vals_hbm.at[pl.ds(base, CHUNK)], m_v.at[b], vsem).wait())
            quad(lambda b, base: pltpu.sync_copy(
                m_v.at[b], acc_sh.at[idx_v.at[b]], add=True))
            return carry

        lax.fori_loop(0, (ITERS + PIPE - 1) // PIPE, body, 0)
        plsc.subcore_barrier()

        @pl.when(sid == 0)
        def _():
            pltpu.sync_copy(acc_sh, out_hbm.at[cid])

    return k(vals, dst)


def _tc_node1(p, xroot, gcn_W):
    def body(p_ref, xr_ref, gw_ref, out_ref):
        agg = p_ref[0, :, 0:HID] + p_ref[1, :, 0:HID]
        deg = p_ref[0, :, HID:HID + 1] + p_ref[1, :, HID:HID + 1] + 1.0
        h1 = jnp.maximum(agg + xr_ref[...], 0.0)
        dinv = lax.rsqrt(deg)
        xw = jnp.dot(h1, gw_ref[...], preferred_element_type=jnp.float32)
        xwd = xw * dinv
        out_ref[...] = jnp.concatenate(
            [xwd, dinv, jnp.zeros((N, W - HID - 1), jnp.float32)], axis=1)

    return pl.pallas_call(
        body,
        out_shape=jax.ShapeDtypeStruct((N, W), jnp.float32),
    )(p, xroot, gcn_W)


def _sc_gather_scatter(xwd, src, dst):
    @functools.partial(
        pl.kernel,
        mesh=_mesh(),
        compiler_params=_SC_PARAMS,
        out_type=jax.ShapeDtypeStruct((NC, N, W), jnp.float32),
        scratch_types=[
            pltpu.VMEM((PIPE, CHUNK), jnp.int32),
            pltpu.VMEM((PIPE, CHUNK), jnp.int32),
            pltpu.VMEM((PIPE, CHUNK, W), jnp.float32),
            pltpu.VMEM((ROWS_PER_TILE, W), jnp.float32),
            pltpu.VMEM_SHARED((N, W), jnp.float32),
            pltpu.SemaphoreType.DMA,
            pltpu.SemaphoreType.DMA,
            pltpu.SemaphoreType.DMA,
        ],
    )
    def k(xwd_hbm, src_hbm, dst_hbm, out_hbm, is_v, id_v, rows_v, z_v, acc_sh,
          ssem, dsem, gsem):
        cid = lax.axis_index("c")
        sid = lax.axis_index("s")
        wid = sid * NC + cid

        _zero_spmem(z_v, acc_sh, sid)
        plsc.subcore_barrier()

        def body(i, carry):
            def quad(stage):
                for b in range(PIPE):
                    c = wid + (i * PIPE + b) * NW

                    @pl.when(c < NUM_CHUNKS)
                    def _():
                        stage(b, c * CHUNK)

            quad(lambda b, base: pltpu.async_copy(
                src_hbm.at[pl.ds(base, CHUNK)], is_v.at[b], ssem))
            quad(lambda b, base: pltpu.async_copy(
                dst_hbm.at[pl.ds(base, CHUNK)], id_v.at[b], dsem))
            quad(lambda b, base: pltpu.make_async_copy(
                src_hbm.at[pl.ds(base, CHUNK)], is_v.at[b], ssem).wait())
            quad(lambda b, base: pltpu.async_copy(
                xwd_hbm.at[is_v.at[b]], rows_v.at[b], gsem))
            quad(lambda b, base: pltpu.make_async_copy(
                dst_hbm.at[pl.ds(base, CHUNK)], id_v.at[b], dsem).wait())
            quad(lambda b, base: pltpu.make_async_copy(
                xwd_hbm.at[is_v.at[b]], rows_v.at[b], gsem).wait())
            quad(lambda b, base: pltpu.sync_copy(
                rows_v.at[b], acc_sh.at[id_v.at[b]], add=True))
            return carry

        lax.fori_loop(0, (ITERS + PIPE - 1) // PIPE, body, 0)
        plsc.subcore_barrier()

        @pl.when(sid == 0)
        def _():
            pltpu.sync_copy(acc_sh, out_hbm.at[cid])

    return k(xwd, src, dst)


def _tc_node2(p2, xwd, gcn_br, lin_WT, lin_br):
    def body(p_ref, xwd_ref, gb_ref, lw_ref, lb_ref, out_ref):
        dinv = xwd_ref[:, HID:HID + 1]
        agg2 = p_ref[0, :, 0:HID] + p_ref[1, :, 0:HID] + xwd_ref[:, 0:HID]
        h2 = agg2 * dinv + gb_ref[...]
        wu = lw_ref[:, 0:HID]
        wv = lw_ref[:, HID:2 * HID]
        u = jnp.sum(h2 * wu, axis=1, keepdims=True) + lb_ref[...]
        v = jnp.sum(h2 * wv, axis=1, keepdims=True)
        out_ref[...] = jnp.concatenate(
            [u, v, jnp.zeros((N, 14), jnp.float32)], axis=1)

    return pl.pallas_call(
        body,
        out_shape=jax.ShapeDtypeStruct((N, 16), jnp.float32),
    )(p2, xwd, gcn_br, lin_WT, lin_br)


def _sc_edge_scores(u, v, src, dst):
    @functools.partial(
        pl.kernel,
        mesh=_mesh(),
        compiler_params=_SC_PARAMS,
        out_type=jax.ShapeDtypeStruct((E,), jnp.float32),
        scratch_types=[
            pltpu.VMEM((N,), jnp.float32),
            pltpu.VMEM((N,), jnp.float32),
            pltpu.VMEM((PIPE, CHUNK), jnp.int32),
            pltpu.VMEM((PIPE, CHUNK), jnp.int32),
            pltpu.VMEM((PIPE, CHUNK), jnp.float32),
            pltpu.SemaphoreType.DMA,
            pltpu.SemaphoreType.DMA,
            pltpu.SemaphoreType.DMA,
        ],
    )
    def k(u_hbm, v_hbm, src_hbm, dst_hbm, out_hbm, u_v, v_v, is_v, id_v, o_v,
          ssem, dsem, osem):
        wid = lax.axis_index("s") * NC + lax.axis_index("c")
        pltpu.sync_copy(u_hbm, u_v)
        pltpu.sync_copy(v_hbm, v_v)

        def body(i, carry):
            def quad(stage):
                for b in range(PIPE):
                    c = wid + (i * PIPE + b) * NW

                    @pl.when(c < NUM_CHUNKS)
                    def _():
                        stage(b, c * CHUNK)

            quad(lambda b, base: pltpu.async_copy(
                src_hbm.at[pl.ds(base, CHUNK)], is_v.at[b], ssem))
            quad(lambda b, base: pltpu.async_copy(
                dst_hbm.at[pl.ds(base, CHUNK)], id_v.at[b], dsem))
            quad(lambda b, base: pltpu.make_async_copy(
                src_hbm.at[pl.ds(base, CHUNK)], is_v.at[b], ssem).wait())
            quad(lambda b, base: pltpu.make_async_copy(
                dst_hbm.at[pl.ds(base, CHUNK)], id_v.at[b], dsem).wait())

            def compute(b, base):
                for g in range(CHUNK // 16):
                    s16 = is_v[b, pl.ds(g * 16, 16)]
                    d16 = id_v[b, pl.ds(g * 16, 16)]
                    us = plsc.load_gather(u_v, [s16])
                    vd = plsc.load_gather(v_v, [d16])
                    o_v[b, pl.ds(g * 16, 16)] = us + vd
                pltpu.async_copy(o_v.at[b], out_hbm.at[pl.ds(base, CHUNK)],
                                 osem)

            quad(compute)
            quad(lambda b, base: pltpu.make_async_copy(
                o_v.at[b], out_hbm.at[pl.ds(base, CHUNK)], osem).wait())
            return carry

        lax.fori_loop(0, (ITERS + PIPE - 1) // PIPE, body, 0)

    return k(u, v, src, dst)


def _tc_sigmoid(scores2d):
    def body(s_ref, out_ref):
        out_ref[...] = jax.nn.sigmoid(s_ref[...])

    return pl.pallas_call(
        body,
        out_shape=jax.ShapeDtypeStruct(scores2d.shape, jnp.float32),
    )(scores2d)


def kernel(x, edge_index, edge_attr, W1, b1, W2, b2, root, conv1_bias,
           gcn_W, gcn_b, lin_W, lin_b):
    src = edge_index[0]
    dst = edge_index[1]
    W2p = (W2.reshape(H2, IN_C, HID).transpose(0, 2, 1).reshape(H2, H2)
           .astype(jnp.bfloat16))
    b2pr = b2.reshape(IN_C, HID).T.reshape(1, H2)
    b1r = b1.reshape(1, H2)
    c1br = conv1_bias.reshape(1, HID)
    gcn_br = gcn_b.reshape(1, HID)
    lin_WT = lin_W.reshape(2 * HID, 1).T
    lin_br = lin_b.reshape(1, 1)

    xj = _sc_gather_x(x, src)
    msgc, xroot = _tc_nnconv(edge_attr, xj, W1, b1r, W2p, b2pr, x, root, c1br)
    p = _sc_scatter_rows(msgc, dst)
    xwd = _tc_node1(p, xroot, gcn_W)
    p2 = _sc_gather_scatter(xwd, src, dst)
    uv = _tc_node2(p2, xwd, gcn_br, lin_WT, lin_br)
    u = uv[:, 0]
    v = uv[:, 1]
    scores = _sc_edge_scores(u, v, src, dst)
    return _tc_sigmoid(scores.reshape(E // 128, 128)).reshape(E)
